# baseline (speedup 1.0000x reference)
import functools

import jax
import jax.numpy as jnp
from jax import lax
from jax.experimental import pallas as pl
from jax.experimental.pallas import tpu as pltpu
from jax.experimental.pallas import tpu_sc as plsc

_N_FRAGS = 16384
_IN_DIM = 4096
_N_EXPERTS = 64
_TOP_K = 8
_NCHUNK = 4
_CROWS = _N_FRAGS // _NCHUNK
_BLOCK_COLS = 1024
_GRID = _CROWS // _BLOCK_COLS
_LANES = 16
_NW = 32
_RPW = _CROWS // _NW
_NGROUPS = _RPW // _LANES


def _logits_block(x_ref, w_ref, out_ref):
    out_ref[...] = jax.lax.dot_general(
        w_ref[...].astype(jnp.bfloat16),
        x_ref[...].astype(jnp.bfloat16),
        dimension_numbers=(((1,), (1,)), ((), ())),
        preferred_element_type=jnp.float32,
    )


def _tc_logits_t(xc, W):
    return pl.pallas_call(
        _logits_block,
        grid=(_GRID,),
        in_specs=[
            pl.BlockSpec((_BLOCK_COLS, _IN_DIM), lambda i: (i, 0)),
            pl.BlockSpec((_N_EXPERTS, _IN_DIM), lambda i: (0, 0)),
        ],
        out_specs=pl.BlockSpec((_N_EXPERTS, _BLOCK_COLS), lambda i: (0, i)),
        out_shape=jax.ShapeDtypeStruct((_N_EXPERTS, _CROWS), jnp.float32),
        compiler_params=pltpu.CompilerParams(
            dimension_semantics=("parallel",),
        ),
    )(xc, W)


_MESH = plsc.VectorSubcoreMesh(core_axis_name="c", subcore_axis_name="s")


@functools.partial(
    pl.kernel,
    mesh=_MESH,
    out_type=[
        jax.ShapeDtypeStruct((_N_EXPERTS, _CROWS), jnp.float32),
        jax.ShapeDtypeStruct((_NW, _N_EXPERTS, _LANES), jnp.float32),
    ],
    scratch_types=[
        pltpu.VMEM((_N_EXPERTS, _RPW), jnp.float32),
        pltpu.VMEM((_N_EXPERTS, _RPW), jnp.float32),
        pltpu.VMEM((_N_EXPERTS, _LANES), jnp.float32),
    ],
)
def _sc_route(lt_hbm, wt_hbm, part_hbm, lbuf, wbuf, accbuf):
    wid = lax.axis_index("s") * 2 + lax.axis_index("c")
    base = wid * _RPW
    pltpu.sync_copy(lt_hbm.at[:, pl.ds(base, _RPW)], lbuf)

    neg_inf = jnp.float32(-jnp.inf)
    zeros = jnp.zeros((_LANES,), jnp.float32)

    for e in range(_N_EXPERTS):
        accbuf[e] = zeros

    def group(g, carry):
        sl = pl.ds(g * _LANES, _LANES)
        work = [lbuf[e, sl] for e in range(_N_EXPERTS)]
        row_max = None
        thresh = None
        for t in range(_TOP_K):
            m = work[0]
            for e in range(1, _N_EXPERTS):
                m = jnp.maximum(m, work[e])
            if t == 0:
                row_max = m
            thresh = m
            if t < _TOP_K - 1:
                for e in range(_N_EXPERTS):
                    work[e] = jnp.where(work[e] == m, neg_inf, work[e])
        s = zeros
        ev = []
        for e in range(_N_EXPERTS):
            le = lbuf[e, sl]
            x = jnp.where(le >= thresh, jnp.exp(le - row_max), 0.0)
            ev.append(x)
            s = s + x
        inv = 1.0 / s
        for e in range(_N_EXPERTS):
            w = ev[e] * inv
            wbuf[e, sl] = w
            accbuf[e] = accbuf[e] + w
        return carry

    lax.fori_loop(0, _NGROUPS, group, 0)

    pltpu.sync_copy(wbuf, wt_hbm.at[:, pl.ds(base, _RPW)])
    pltpu.sync_copy(accbuf, part_hbm.at[wid])


@functools.partial(jax.jit)
def kernel(x, W):
    xc = x.reshape(_NCHUNK, _CROWS, _IN_DIM)
    wts = []
    parts = []
    for c in range(_NCHUNK):
        lt = _tc_logits_t(xc[c], W)
        wtc, pc = _sc_route(lt)
        wts.append(wtc)
        parts.append(pc)
    weights = jnp.concatenate(wts, axis=1).T
    load = sum(p.sum(axis=(0, 2)) for p in parts) * (1.0 / _N_FRAGS)
    return weights, load


# speedup vs baseline: 3.0472x; 3.0472x over previous
import functools

import jax
import jax.numpy as jnp
from jax.experimental import pallas as pl
from jax.experimental.pallas import tpu as pltpu

_N_FRAGS = 16384
_IN_DIM = 4096
_N_EXPERTS = 64
_TOP_K = 8
_BLOCK_ROWS = 1024
_GRID = _N_FRAGS // _BLOCK_ROWS


def _router_block(x_ref, wt_ref, w_out_ref, part_ref):
    logits = jnp.dot(
        x_ref[...].astype(jnp.bfloat16),
        wt_ref[...].astype(jnp.bfloat16),
        preferred_element_type=jnp.float32,
    )

    work = logits
    sel = jnp.zeros(logits.shape, dtype=jnp.bool_)
    row_max = None
    for t in range(_TOP_K):
        m = jnp.max(work, axis=-1, keepdims=True)
        if t == 0:
            row_max = m
        hit = work == m
        sel = jnp.logical_or(sel, hit)
        work = jnp.where(hit, -jnp.inf, work)

    e = jnp.where(sel, jnp.exp(logits - row_max), 0.0)
    weights = e / jnp.sum(e, axis=-1, keepdims=True)
    w_out_ref[...] = weights
    part_ref[...] = jnp.sum(weights, axis=0, keepdims=True)[None] * (1.0 / _N_FRAGS)


@functools.partial(jax.jit)
def kernel(x, W):
    wt = W.T
    weights, parts = pl.pallas_call(
        _router_block,
        grid=(_GRID,),
        in_specs=[
            pl.BlockSpec((_BLOCK_ROWS, _IN_DIM), lambda i: (i, 0)),
            pl.BlockSpec((_IN_DIM, _N_EXPERTS), lambda i: (0, 0)),
        ],
        out_specs=[
            pl.BlockSpec((_BLOCK_ROWS, _N_EXPERTS), lambda i: (i, 0)),
            pl.BlockSpec((1, 1, _N_EXPERTS), lambda i: (i, 0, 0)),
        ],
        out_shape=[
            jax.ShapeDtypeStruct((_N_FRAGS, _N_EXPERTS), jnp.float32),
            jax.ShapeDtypeStruct((_GRID, 1, _N_EXPERTS), jnp.float32),
        ],
        compiler_params=pltpu.CompilerParams(
            dimension_semantics=("parallel",),
        ),
    )(x, wt)
    return weights, parts.sum(axis=(0, 1))
